# baseline (speedup 1.0000x reference)
import functools

import jax
import jax.numpy as jnp
from jax import lax
from jax.experimental import pallas as pl
from jax.experimental.pallas import tpu as pltpu
from jax.experimental.pallas import tpu_sc as plsc

_L = 16


def _xlane_sum(v):
    lanes = lax.iota(jnp.int32, _L)
    for sh in (8, 4, 2, 1):
        idx = (lanes + sh) & (_L - 1)
        v = v + v.at[idx].get(mode="promise_in_bounds")
    return v


def _rsqrt(x):
    i = lax.bitcast_convert_type(x, jnp.int32)
    y = lax.bitcast_convert_type(jnp.int32(0x5F3759DF) - (i >> 1), jnp.float32)
    for _ in range(3):
        y = y * (1.5 - 0.5 * x * y * y)
    return y


def kernel(input_ids, token_table, pos_table, ln_gamma, ln_beta):
    B, S = input_ids.shape
    V, H = token_table.shape
    n_vec = H // _L

    info = plsc.get_sparse_core_info()
    NC, NS = info.num_cores, info.num_subcores
    NW = NC * NS
    P = S // NW
    C = 16
    n_chunks = P // C

    mesh = plsc.VectorSubcoreMesh(core_axis_name="c", subcore_axis_name="s")

    @functools.partial(
        pl.kernel,
        mesh=mesh,
        out_type=jax.ShapeDtypeStruct((B, S, H), jnp.float32),
        scratch_types=[
            pltpu.VMEM((B, C), jnp.int32),
            pltpu.VMEM((B, C, H), jnp.float32),
            pltpu.VMEM((C, H), jnp.float32),
            pltpu.VMEM((H,), jnp.float32),
            pltpu.VMEM((H,), jnp.float32),
            pltpu.SemaphoreType.DMA,
        ],
    )
    def emb_ln(ids_hbm, tok_hbm, pos_hbm, gam_hbm, bet_hbm, out_hbm,
               idx_v, rows_v, pos_v, gam_v, bet_v, sem):
        wid = lax.axis_index("s") * NC + lax.axis_index("c")
        p_base = wid * P
        pltpu.sync_copy(gam_hbm, gam_v)
        pltpu.sync_copy(bet_hbm, bet_v)

        def chunk_body(ci, _):
            p0 = p_base + ci * C
            for b in range(B):
                pltpu.sync_copy(ids_hbm.at[b, pl.ds(p0, C)], idx_v.at[b])
            cps = [pltpu.async_copy(pos_hbm.at[pl.ds(p0, C), :], pos_v, sem)]
            for b in range(B):
                cps.append(
                    pltpu.async_copy(tok_hbm.at[idx_v.at[b]], rows_v.at[b], sem))
            for cp in cps:
                cp.wait()

            for b in range(B):
                def tok_body(t, _, b=b):
                    zero = jnp.zeros((_L,), jnp.float32)

                    def acc_body(i, carry):
                        s, q = carry
                        off = i * _L
                        v = (rows_v[b, t, pl.ds(off, _L)]
                             + pos_v[t, pl.ds(off, _L)])
                        rows_v[b, t, pl.ds(off, _L)] = v
                        return (s + v, q + v * v)

                    s, q = lax.fori_loop(0, n_vec, acc_body, (zero, zero))
                    mean = _xlane_sum(s) * (1.0 / H)
                    var = _xlane_sum(q) * (1.0 / H) - mean * mean
                    r = _rsqrt(var + 1e-5)

                    def norm_body(i, _):
                        off = i * _L
                        xv = rows_v[b, t, pl.ds(off, _L)]
                        rows_v[b, t, pl.ds(off, _L)] = (
                            (xv - mean) * r * gam_v[pl.ds(off, _L)]
                            + bet_v[pl.ds(off, _L)])
                        return 0

                    lax.fori_loop(0, n_vec, norm_body, 0)
                    return 0

                lax.fori_loop(0, C, tok_body, 0)

            for b in range(B):
                pltpu.sync_copy(rows_v.at[b], out_hbm.at[b, pl.ds(p0, C), :])
            return 0

        lax.fori_loop(0, n_chunks, chunk_body, 0)

    return emb_ln(input_ids.astype(jnp.int32), token_table, pos_table,
                  ln_gamma, ln_beta)
